# baseline (speedup 1.0000x reference)
import functools

import jax
import jax.numpy as jnp
from jax import lax
from jax.experimental import pallas as pl
from jax.experimental.pallas import tpu as pltpu
from jax.experimental.pallas import tpu_sc as plsc

B = 64
N = 32768
TWO_N = 2 * N
M = B * N
NC = 2
NS = 16
NW = NC * NS
BPW = B // NW
C = 4096
NCH = N // C
U = 8
GROUPS = C // 16


def _sl1_pair(d0, d1):
    ad0 = jnp.abs(d0)
    ad1 = jnp.abs(d1)
    m0 = jnp.minimum(ad0, 1.0)
    m1 = jnp.minimum(ad1, 1.0)
    return (ad0 + ad1) - (m0 + m1) + 0.5 * (m0 * m0 + m1 * m1)


_mesh = plsc.VectorSubcoreMesh(core_axis_name="c", subcore_axis_name="s")


@functools.partial(
    pl.kernel,
    out_type=jax.ShapeDtypeStruct((NW, 3, 16), jnp.float32),
    mesh=_mesh,
    compiler_params=pltpu.CompilerParams(needs_layout_passes=False),
    scratch_types=[
        pltpu.VMEM((TWO_N,), jnp.float32),
        pltpu.VMEM((2, 4, C), jnp.float32),
        pltpu.VMEM((3, 16), jnp.float32),
        pltpu.SemaphoreType.DMA,
        pltpu.SemaphoreType.DMA,
        pltpu.SemaphoreType.DMA,
    ],
)
def _partials(x_hbm, t_hbm, out_hbm, xrow, tbuf, res, sem0, sem1, xsem):
    cid = lax.axis_index("c")
    sid = lax.axis_index("s")
    wid = sid * NC + cid
    iota = lax.broadcasted_iota(jnp.int32, (16,), 0)
    zeros = jnp.zeros((16,), jnp.float32)
    ones = jnp.ones((16,), jnp.float32)
    sems = (sem0, sem1)

    def row_group(s, base):
        t0 = tbuf[s, 0, pl.ds(base, 16)]
        t1 = tbuf[s, 1, pl.ds(base, 16)]
        ti = tbuf[s, 2, pl.ds(base, 16)]
        st = tbuf[s, 3, pl.ds(base, 16)]
        idx = ti.astype(jnp.int32)
        xlo = plsc.load_gather(xrow, [idx])
        xhi = plsc.load_gather(xrow, [idx + N])
        return _sl1_pair(xlo - t0, xhi - t1), st == 1.0

    def fire(b, c, s):
        return [
            pltpu.async_copy(t_hbm.at[j, b, pl.ds(c * C, C)], tbuf.at[s, j], sems[s])
            for j in range(4)
        ]

    acc = zeros
    cnt = zeros
    res[2] = zeros
    for i in range(BPW):
        b = wid * BPW + i
        xcopy = pltpu.async_copy(x_hbm.at[b], xrow, xsem)
        pending = fire(b, 0, 0)
        xcopy.wait()
        for c in range(NCH):
            s = c % 2
            nxt = fire(b, c + 1, 1 - s) if c + 1 < NCH else []
            for h in pending:
                h.wait()
            pending = nxt

            if i == 0 and c == 0:
                @pl.when(wid == 0)
                def _():
                    pair, _ = row_group(0, 0)
                    res[2] = jnp.where(iota == 0, pair, zeros)

            @plsc.parallel_loop(0, C, 16, unroll=U, carry=(acc, cnt))
            def _loop(base, carry, s=s):
                acc, cnt = carry
                pair, m = row_group(s, base)
                w = jnp.where(m, ones, zeros)
                return acc + pair * w, cnt + w

            acc, cnt = _loop

    res[0] = acc
    res[1] = cnt
    pltpu.sync_copy(res, out_hbm.at[wid])


def kernel(input, target):
    t4 = jnp.transpose(jnp.reshape(target, (B, N, 4)), (2, 0, 1))
    parts = _partials(input, t4)
    s = jnp.sum(parts[:, 0, :])
    c = jnp.sum(parts[:, 1, :])
    p00 = parts[0, 2, 0]
    return (s + (jnp.float32(M) - c) * p00) / jnp.float32(M)


# speedup vs baseline: 1.0218x; 1.0218x over previous
import functools

import jax
import jax.numpy as jnp
from jax import lax
from jax.experimental import pallas as pl
from jax.experimental.pallas import tpu as pltpu
from jax.experimental.pallas import tpu_sc as plsc

B = 64
N = 32768
TWO_N = 2 * N
M = B * N
NC = 2
NS = 16
NW = NC * NS
BPW = B // NW
C = 2048
NCH = N // C
NBUF = 4
U = 8


def _sl1_pair(d0, d1):
    ad0 = jnp.abs(d0)
    ad1 = jnp.abs(d1)
    m0 = jnp.minimum(ad0, 1.0)
    m1 = jnp.minimum(ad1, 1.0)
    return (ad0 + ad1) - (m0 + m1) + 0.5 * (m0 * m0 + m1 * m1)


_mesh = plsc.VectorSubcoreMesh(core_axis_name="c", subcore_axis_name="s")


@functools.partial(
    pl.kernel,
    out_type=jax.ShapeDtypeStruct((NW, 3, 16), jnp.float32),
    mesh=_mesh,
    compiler_params=pltpu.CompilerParams(needs_layout_passes=False),
    scratch_types=[
        pltpu.VMEM((TWO_N,), jnp.float32),
        pltpu.VMEM((NBUF, 4, C), jnp.float32),
        pltpu.VMEM((3, 16), jnp.float32),
        pltpu.SemaphoreType.DMA,
        pltpu.SemaphoreType.DMA,
        pltpu.SemaphoreType.DMA,
        pltpu.SemaphoreType.DMA,
        pltpu.SemaphoreType.DMA,
    ],
)
def _partials(x_hbm, t_hbm, out_hbm, xrow, tbuf, res, sem0, sem1, sem2, sem3, xsem):
    cid = lax.axis_index("c")
    sid = lax.axis_index("s")
    wid = sid * NC + cid
    iota = lax.broadcasted_iota(jnp.int32, (16,), 0)
    zeros = jnp.zeros((16,), jnp.float32)
    ones = jnp.ones((16,), jnp.float32)
    sems = (sem0, sem1, sem2, sem3)

    def row_group(k, base, ref=None):
        tb = tbuf if ref is None else ref
        t0 = tb[k, 0, pl.ds(base, 16)]
        t1 = tb[k, 1, pl.ds(base, 16)]
        ti = tb[k, 2, pl.ds(base, 16)]
        st = tb[k, 3, pl.ds(base, 16)]
        idx = ti.astype(jnp.int32)
        xlo = plsc.load_gather(xrow, [idx])
        xhi = plsc.load_gather(xrow, [idx + N])
        return _sl1_pair(xlo - t0, xhi - t1), st == 1.0

    def fire(b, c, k):
        for j in range(4):
            pltpu.async_copy(t_hbm.at[j, b, pl.ds(c * C, C)], tbuf.at[k, j], sems[k])

    def drain(k):
        for j in range(4):
            pltpu.make_async_copy(t_hbm.at[j, 0, pl.ds(0, C)], tbuf.at[k, j], sems[k]).wait()

    acc = zeros
    cnt = zeros
    res[2] = zeros
    XS = TWO_N // 4
    for i in range(BPW):
        b = wid * BPW + i
        xcopies = [
            pltpu.async_copy(x_hbm.at[b, pl.ds(q * XS, XS)], xrow.at[pl.ds(q * XS, XS)], xsem)
            for q in range(4)
        ]
        for k in range(NBUF - 1):
            fire(b, k, k)
        for h in xcopies:
            h.wait()

        if i == 0:
            @pl.when(wid == 0)
            def _():
                for j in range(4):
                    pltpu.sync_copy(
                        t_hbm.at[j, 0, pl.ds(0, 16)],
                        tbuf.at[NBUF - 1, j, pl.ds(0, 16)],
                    )
                pair, _ = row_group(NBUF - 1, 0)
                res[2] = jnp.where(iota == 0, pair, zeros)

        fire(b, NBUF - 1, NBUF - 1)

        def outer_body(t, carry, b=b):
            acc, cnt = carry
            for k in range(NBUF):
                cc = t * NBUF + k
                drain(k)

                @plsc.parallel_loop(0, C, 16, unroll=U, carry=(acc, cnt))
                def _loop(base, carry, k=k):
                    a, n = carry
                    pair, m = row_group(k, base)
                    w = jnp.where(m, ones, zeros)
                    return a + pair * w, n + w

                acc, cnt = _loop
                nxt = cc + NBUF

                @pl.when(nxt < NCH)
                def _(nxt=nxt, k=k, b=b):
                    fire(b, nxt, k)

            return acc, cnt

        acc, cnt = lax.fori_loop(0, NCH // NBUF, outer_body, (acc, cnt))

    res[0] = acc
    res[1] = cnt
    pltpu.sync_copy(res, out_hbm.at[wid])


def kernel(input, target):
    t4 = jnp.transpose(jnp.reshape(target, (B, N, 4)), (2, 0, 1))
    parts = _partials(input, t4)
    s = jnp.sum(parts[:, 0, :])
    c = jnp.sum(parts[:, 1, :])
    p00 = parts[0, 2, 0]
    return (s + (jnp.float32(M) - c) * p00) / jnp.float32(M)
